# Initial kernel scaffold; baseline (speedup 1.0000x reference)
import jax
import jax.numpy as jnp
from jax.experimental import pallas as pl


def kernel(x, Wq, Wk, Wv, Wo):
    raise NotImplementedError("write your pallas kernel here")


# speedup vs baseline: 8.9329x; 8.9329x over previous
import numpy as np
import jax
import jax.numpy as jnp
from jax.experimental import pallas as pl

_H = 12
_DH = 64
_K = 256
_BQ = 256

_INT_MIN = np.int32(-(2 ** 31))
_INT_LOW31 = np.int32(0x7FFFFFFF)


def _proj_kernel(x_ref, w_ref, o_ref):
    o_ref[...] = jnp.dot(x_ref[...], w_ref[...],
                         preferred_element_type=jnp.float32)


def _attn_kernel(q_ref, k_ref, v_ref, o_ref):
    q = q_ref[0]
    k = k_ref[0]
    s = jnp.dot(q, k.T, preferred_element_type=jnp.float32) * (1.0 / 8.0)

    bits = jax.lax.bitcast_convert_type(s, jnp.int32)
    key = jnp.where(bits < 0, bits ^ _INT_LOW31, bits)

    bq = q.shape[0]
    t = jnp.zeros((bq, 1), jnp.int32)
    for b in range(31, -1, -1):
        m = (1 << b) if b < 31 else ((1 << 31) - (1 << 32))
        t_try = t | np.int32(m)
        st = t_try ^ _INT_MIN
        cnt = jnp.sum((key >= st).astype(jnp.int32), axis=1, keepdims=True)
        t = jnp.where(cnt >= _K, t_try, t)
    skt = t ^ _INT_MIN

    keep = key >= skt
    sm = jnp.where(keep, s, jnp.float32(-1e30))
    mx = jnp.max(sm, axis=1, keepdims=True)
    p = jnp.exp(sm - mx)
    l = jnp.sum(p, axis=1, keepdims=True)
    o = jnp.dot(p, v_ref[0], preferred_element_type=jnp.float32) / l
    o_ref[0] = o


def kernel(x, Wq, Wk, Wv, Wo):
    B, S, D = x.shape
    x2 = x.reshape(S, D)
    Wqkv = jnp.concatenate([Wq, Wk, Wv], axis=1)

    qkv = pl.pallas_call(
        _proj_kernel,
        grid=(S // _BQ,),
        in_specs=[pl.BlockSpec((_BQ, D), lambda i: (i, 0)),
                  pl.BlockSpec((D, 3 * D), lambda i: (0, 0))],
        out_specs=pl.BlockSpec((_BQ, 3 * D), lambda i: (i, 0)),
        out_shape=jax.ShapeDtypeStruct((S, 3 * D), jnp.float32),
    )(x2, Wqkv)

    q = qkv[:, :D].reshape(S, _H, _DH).transpose(1, 0, 2)
    k = qkv[:, D:2 * D].reshape(S, _H, _DH).transpose(1, 0, 2)
    v = qkv[:, 2 * D:].reshape(S, _H, _DH).transpose(1, 0, 2)

    attn = pl.pallas_call(
        _attn_kernel,
        grid=(_H, S // _BQ),
        in_specs=[pl.BlockSpec((1, _BQ, _DH), lambda h, i: (h, i, 0)),
                  pl.BlockSpec((1, S, _DH), lambda h, i: (h, 0, 0)),
                  pl.BlockSpec((1, S, _DH), lambda h, i: (h, 0, 0))],
        out_specs=pl.BlockSpec((1, _BQ, _DH), lambda h, i: (h, i, 0)),
        out_shape=jax.ShapeDtypeStruct((_H, S, _DH), jnp.float32),
    )(q, k, v)
    attn = attn.transpose(1, 0, 2).reshape(S, D)

    out = pl.pallas_call(
        _proj_kernel,
        grid=(S // _BQ,),
        in_specs=[pl.BlockSpec((_BQ, D), lambda i: (i, 0)),
                  pl.BlockSpec((D, D), lambda i: (0, 0))],
        out_specs=pl.BlockSpec((_BQ, D), lambda i: (i, 0)),
        out_shape=jax.ShapeDtypeStruct((S, D), jnp.float32),
    )(attn, Wo)

    return out.reshape(B, S, D)
